# baseline (speedup 1.0000x reference)
import functools

import jax
import jax.numpy as jnp
from jax import lax
from jax.experimental import pallas as pl
from jax.experimental.pallas import tpu as pltpu
from jax.experimental.pallas import tpu_sc as plsc

NC, NS = 2, 16
NW = NC * NS
GI = 128


def _sc_lookup(B, C, EMB, E1f, E2f, idx2d, dst2d, f0dst, xvf, xi0, xv0, wb):
  CF = C - 1
  SPT = B // NW
  CS = 64
  CPT = SPT // CS
  RC = CS * CF
  NG = RC // GI
  F0G = SPT // GI
  assert RC % GI == 0 and SPT % CS == 0 and SPT % GI == 0

  mesh = plsc.VectorSubcoreMesh(core_axis_name="c", subcore_axis_name="s")

  @functools.partial(
      pl.kernel,
      out_type=(
          jax.ShapeDtypeStruct((B * C, EMB), jnp.float32),
          jax.ShapeDtypeStruct((B * C, EMB), jnp.float32),
      ),
      mesh=mesh,
      compiler_params=pltpu.CompilerParams(use_tc_tiling_on_sc=False),
      scratch_types=[
          pltpu.VMEM((NG, GI), jnp.int32),
          pltpu.VMEM((NG, GI), jnp.int32),
          pltpu.VMEM((F0G, GI), jnp.int32),
          pltpu.VMEM((RC,), jnp.float32),
          pltpu.VMEM((RC, EMB), jnp.float32),
          pltpu.VMEM((RC, EMB), jnp.float32),
          pltpu.VMEM((SPT,), jnp.float32),
          pltpu.VMEM((SPT,), jnp.float32),
          pltpu.VMEM((SPT, EMB), jnp.float32),
          pltpu.VMEM((SPT, EMB), jnp.float32),
          pltpu.VMEM((4, EMB), jnp.float32),
          pltpu.SemaphoreType.DMA,
      ],
  )
  def sck(e1r, e2r, idxr, dstr, f0dr, xvr, xi0r, xv0r, wbr, out1r, out2r,
          idx_v, dst_v, f0d_v, xv_v, buf1, buf2, xi0_v, xv0_v,
          f0b1, f0b2, wb_v, sem):
    wid = lax.axis_index("s") * NC + lax.axis_index("c")
    s0 = wid * SPT

    pltpu.sync_copy(wbr, wb_v)
    pltpu.sync_copy(xi0r.at[pl.ds(s0, SPT)], xi0_v)
    pltpu.sync_copy(xv0r.at[pl.ds(s0, SPT)], xv0_v)
    pltpu.sync_copy(f0dr.at[pl.ds(wid * F0G, F0G)], f0d_v)
    w1 = wb_v[0]
    c1 = wb_v[1]
    w2 = wb_v[2]
    c2 = wb_v[3]

    @plsc.parallel_loop(0, SPT, step=16)
    def _f0(j):
      xiv = xi0_v[pl.ds(j, 16)]
      xvv = xv0_v[pl.ds(j, 16)]
      for l in range(16):
        f0b1[j + l] = (w1 * xiv[l] + c1) * xvv[l]
        f0b2[j + l] = (w2 * xiv[l] + c2) * xvv[l]

    f0cps = []
    for g in range(F0G):
      f0cps.append(pltpu.async_copy(
          f0b1.at[pl.ds(g * GI, GI)], out1r.at[f0d_v.at[g]], sem))
      f0cps.append(pltpu.async_copy(
          f0b2.at[pl.ds(g * GI, GI)], out2r.at[f0d_v.at[g]], sem))
    for cp in f0cps:
      cp.wait()

    def chunk(c, carry):
      sb = s0 + c * CS
      irow = wid * (CPT * NG) + c * NG
      pltpu.sync_copy(idxr.at[pl.ds(irow, NG)], idx_v)
      pltpu.sync_copy(dstr.at[pl.ds(irow, NG)], dst_v)
      pltpu.sync_copy(xvr.at[pl.ds(sb * CF, RC)], xv_v)
      cps = []
      for r in range(NG):
        cps.append(pltpu.async_copy(
            e1r.at[idx_v.at[r]], buf1.at[pl.ds(r * GI, GI)], sem))
        cps.append(pltpu.async_copy(
            e2r.at[idx_v.at[r]], buf2.at[pl.ds(r * GI, GI)], sem))
      for cp in cps:
        cp.wait()

      @plsc.parallel_loop(0, RC, step=16)
      def _scale(j):
        xvv = xv_v[pl.ds(j, 16)]
        for l in range(16):
          s = xvv[l]
          buf1[j + l] = buf1[j + l] * s
          buf2[j + l] = buf2[j + l] * s

      ocps = []
      for r in range(NG):
        ocps.append(pltpu.async_copy(
            buf1.at[pl.ds(r * GI, GI)], out1r.at[dst_v.at[r]], sem))
        ocps.append(pltpu.async_copy(
            buf2.at[pl.ds(r * GI, GI)], out2r.at[dst_v.at[r]], sem))
      for cp in ocps:
        cp.wait()
      return carry

    lax.fori_loop(0, CPT, chunk, 0)

  return sck(E1f, E2f, idx2d, dst2d, f0dst, xvf, xi0, xv0, wb)


@jax.jit
def kernel(Xi, Xv, W1, b1, E1, W2, b2, E2):
  B, L, C, D = Xi.shape
  V, EMB = E1.shape[1], E1.shape[2]
  BL = B * L
  Xif = Xi.reshape(BL, C).astype(jnp.int32)
  Xvf = Xv.reshape(BL, C)
  idx = Xif[:, 1:] + (jnp.arange(C - 1, dtype=jnp.int32) * V)[None, :]
  idx2d = idx.reshape(-1, GI)
  samp = jnp.arange(BL, dtype=jnp.int32)
  dst = samp[:, None] * C + jnp.arange(1, C, dtype=jnp.int32)[None, :]
  dst2d = dst.reshape(-1, GI)
  f0dst = (samp * C).reshape(-1, GI)
  xvf = Xvf[:, 1:].reshape(-1)
  xi0 = Xif[:, 0].astype(jnp.float32)
  xv0 = Xvf[:, 0]
  wb = jnp.stack([W1[:, 0], b1, W2[:, 0], b2])
  out1, out2 = _sc_lookup(BL, C, EMB, E1.reshape(-1, EMB), E2.reshape(-1, EMB),
                          idx2d, dst2d, f0dst, xvf, xi0, xv0, wb)
  fm_first = out1.reshape(B, L, C * EMB)
  fm_second = out2.reshape(BL, C, EMB)
  return fm_first, fm_second


# speedup vs baseline: 2.6417x; 2.6417x over previous
import functools

import jax
import jax.numpy as jnp
from jax import lax
from jax.experimental import pallas as pl
from jax.experimental.pallas import tpu as pltpu
from jax.experimental.pallas import tpu_sc as plsc

NC, NS = 2, 16
NW = NC * NS
CSZ = 8192


def _sc_lookup(B, C, EMB, V, E1t, E2t, XiT, XvT, xi0, wb):
  NP = 2 * C * EMB
  TPT = NP // NW
  CH = B // CSZ
  assert NP % NW == 0 and B % CSZ == 0

  mesh = plsc.VectorSubcoreMesh(core_axis_name="c", subcore_axis_name="s")

  @functools.partial(
      pl.kernel,
      out_type=(
          jax.ShapeDtypeStruct((C * EMB, B), jnp.float32),
          jax.ShapeDtypeStruct((C * EMB, B), jnp.float32),
      ),
      mesh=mesh,
      compiler_params=pltpu.CompilerParams(
          use_tc_tiling_on_sc=False, needs_layout_passes=False),
      scratch_types=[
          pltpu.VMEM((V,), jnp.float32),
          pltpu.VMEM((CSZ,), jnp.int32),
          pltpu.VMEM((CSZ,), jnp.float32),
          pltpu.VMEM((4, EMB), jnp.float32),
      ],
  )
  def sck(e1r, e2r, xitr, xvtr, xi0r, wbr, o1r, o2r,
          plane, idx_v, xv_v, wb_v):
    wid = lax.axis_index("s") * NC + lax.axis_index("c")
    pltpu.sync_copy(wbr, wb_v)

    def task(k, carry):
      p = wid * TPT + k
      o = p // (C * EMB)
      q = p - o * (C * EMB)
      i = q // EMB
      e = q - i * EMB

      esplat = jnp.full((16,), e, jnp.int32)
      wrow = jnp.full((16,), 2 * o, jnp.int32)
      brow = jnp.full((16,), 2 * o + 1, jnp.int32)
      ws = plsc.load_gather(wb_v, [wrow, esplat])
      bs = plsc.load_gather(wb_v, [brow, esplat])

      @pl.when((i > 0) & (o == 0))
      def _():
        pltpu.sync_copy(e1r.at[i - 1, e], plane)

      @pl.when((i > 0) & (o == 1))
      def _():
        pltpu.sync_copy(e2r.at[i - 1, e], plane)

      for cs in range(CH):
        pltpu.sync_copy(xvtr.at[i, pl.ds(cs * CSZ, CSZ)], xv_v)

        @pl.when(i == 0)
        def _():
          pltpu.sync_copy(xi0r.at[pl.ds(cs * CSZ, CSZ)], idx_v)

          @plsc.parallel_loop(0, CSZ, step=16)
          def _f0(j):
            xiv = plsc.bitcast(idx_v[pl.ds(j, 16)], jnp.float32)
            xv_v[pl.ds(j, 16)] = (xiv * ws + bs) * xv_v[pl.ds(j, 16)]

        @pl.when(i > 0)
        def _():
          pltpu.sync_copy(xitr.at[i, pl.ds(cs * CSZ, CSZ)], idx_v)

          @plsc.parallel_loop(0, CSZ, step=16)
          def _gather(j):
            idxv = idx_v[pl.ds(j, 16)]
            vals = plsc.load_gather(plane, [idxv])
            xv_v[pl.ds(j, 16)] = vals * xv_v[pl.ds(j, 16)]

        @pl.when(o == 0)
        def _():
          pltpu.sync_copy(xv_v, o1r.at[q, pl.ds(cs * CSZ, CSZ)])

        @pl.when(o == 1)
        def _():
          pltpu.sync_copy(xv_v, o2r.at[q, pl.ds(cs * CSZ, CSZ)])

      return carry

    lax.fori_loop(0, TPT, task, 0)

  return sck(E1t, E2t, XiT, XvT, xi0, wb)


@jax.jit
def kernel(Xi, Xv, W1, b1, E1, W2, b2, E2):
  B, L, C, D = Xi.shape
  V, EMB = E1.shape[1], E1.shape[2]
  BL = B * L
  XiT = Xi.reshape(BL, C).astype(jnp.int32).T
  XvT = Xv.reshape(BL, C).T
  xi0 = lax.bitcast_convert_type(XiT[0].astype(jnp.float32), jnp.int32)
  wb = jnp.stack([W1[:, 0], b1, W2[:, 0], b2])
  O1, O2 = _sc_lookup(BL, C, EMB, V,
                      E1.transpose(0, 2, 1), E2.transpose(0, 2, 1),
                      XiT, XvT, xi0, wb)
  fm_first = O1.T.reshape(B, L, C * EMB)
  fm_second = O2.reshape(C, EMB, BL).transpose(2, 0, 1)
  return fm_first, fm_second


# speedup vs baseline: 3.2094x; 1.2149x over previous
import functools

import jax
import jax.numpy as jnp
from jax import lax
from jax.experimental import pallas as pl
from jax.experimental.pallas import tpu as pltpu
from jax.experimental.pallas import tpu_sc as plsc

NC, NS = 2, 16
NW = NC * NS
CSZ = 8192


def _sc_table(B, C, EMB, V, Et, XiT, XvT, xi0, wb):
  NP = C * EMB
  TPT = (NP + NW - 1) // NW
  CH = B // CSZ
  assert B % CSZ == 0

  mesh = plsc.VectorSubcoreMesh(core_axis_name="c", subcore_axis_name="s")

  @functools.partial(
      pl.kernel,
      out_type=jax.ShapeDtypeStruct((C * EMB, B), jnp.float32),
      mesh=mesh,
      compiler_params=pltpu.CompilerParams(
          use_tc_tiling_on_sc=False, needs_layout_passes=False),
      scratch_types=[
          pltpu.VMEM((V,), jnp.float32),
          pltpu.VMEM((CSZ,), jnp.int32),
          pltpu.VMEM((CSZ,), jnp.float32),
          pltpu.VMEM((2, EMB), jnp.float32),
      ],
  )
  def sck(er, xitr, xvtr, xi0r, wbr, outr, plane, idx_v, xv_v, wb_v):
    wid = lax.axis_index("s") * NC + lax.axis_index("c")
    pltpu.sync_copy(wbr, wb_v)

    def task(k, carry):
      p = k * NW + wid

      @pl.when(p < NP)
      def _():
        i = p // EMB
        e = p - i * EMB

        esplat = jnp.full((16,), e, jnp.int32)
        zeros = jnp.zeros((16,), jnp.int32)
        ws = plsc.load_gather(wb_v, [zeros, esplat])
        bs = plsc.load_gather(wb_v, [zeros + 1, esplat])

        @pl.when(i > 0)
        def _():
          pltpu.sync_copy(er.at[i - 1, e], plane)

        for cs in range(CH):
          pltpu.sync_copy(xvtr.at[i, pl.ds(cs * CSZ, CSZ)], xv_v)

          @pl.when(i == 0)
          def _():
            pltpu.sync_copy(xi0r.at[pl.ds(cs * CSZ, CSZ)], idx_v)

            @plsc.parallel_loop(0, CSZ, step=16)
            def _f0(j):
              xiv = plsc.bitcast(idx_v[pl.ds(j, 16)], jnp.float32)
              xv_v[pl.ds(j, 16)] = (xiv * ws + bs) * xv_v[pl.ds(j, 16)]

          @pl.when(i > 0)
          def _():
            pltpu.sync_copy(xitr.at[i, pl.ds(cs * CSZ, CSZ)], idx_v)

            @plsc.parallel_loop(0, CSZ, step=16)
            def _gather(j):
              idxv = idx_v[pl.ds(j, 16)]
              vals = plsc.load_gather(plane, [idxv])
              xv_v[pl.ds(j, 16)] = vals * xv_v[pl.ds(j, 16)]

          pltpu.sync_copy(xv_v, outr.at[p, pl.ds(cs * CSZ, CSZ)])

      return carry

    lax.fori_loop(0, TPT, task, 0)

  return sck(Et, XiT, XvT, xi0, wb)


@jax.jit
def kernel(Xi, Xv, W1, b1, E1, W2, b2, E2):
  B, L, C, D = Xi.shape
  V, EMB = E1.shape[1], E1.shape[2]
  BL = B * L
  XiT = Xi.reshape(BL, C).astype(jnp.int32).T
  XvT = Xv.reshape(BL, C).T
  xi0 = lax.bitcast_convert_type(XiT[0].astype(jnp.float32), jnp.int32)
  O1 = _sc_table(BL, C, EMB, V, E1.transpose(0, 2, 1), XiT, XvT, xi0,
                 jnp.stack([W1[:, 0], b1]))
  O2 = _sc_table(BL, C, EMB, V, E2.transpose(0, 2, 1), XiT, XvT, xi0,
                 jnp.stack([W2[:, 0], b2]))
  fm_first = O1.T.reshape(B, L, C * EMB)
  fm_second = O2.reshape(C, EMB, BL).transpose(2, 0, 1)
  return fm_first, fm_second
